# Initial kernel scaffold; baseline (speedup 1.0000x reference)
import jax
import jax.numpy as jnp
from jax.experimental import pallas as pl


def kernel(x, edge_index, W1, b1, W2, b2):
    raise NotImplementedError("write your pallas kernel here")


# speedup vs baseline: 8.0495x; 8.0495x over previous
import functools

import jax
import jax.numpy as jnp
from jax import lax
from jax.experimental import pallas as pl
from jax.experimental.pallas import tpu as pltpu
from jax.experimental.pallas import tpu_sc as plsc

N = 10000
E = 320000
D = 128
NC = 2
NS = 16
NW = NC * NS
EW = E // NW
CH = 125
NCH = EW // CH
ROWS_W = 320
DR = 80
ROWS_T = 640

_MESH = plsc.VectorSubcoreMesh(core_axis_name="c", subcore_axis_name="s")
_SC_PARAMS = pltpu.CompilerParams(needs_layout_passes=False)

f32 = jnp.float32
i32 = jnp.int32


def _wid():
    return lax.axis_index("s") * NC + lax.axis_index("c")


def _rsqrt16(x):
    i = plsc.bitcast(x, i32)
    i = jnp.int32(0x5F3759DF) - (i >> 1)
    y = plsc.bitcast(i, f32)
    for _ in range(3):
        y = y * (1.5 - 0.5 * x * y * y)
    return y


def _norm16(d0, d1):
    return _rsqrt16(jnp.maximum(d0 + d1, 1.0))


def _iota16():
    return lax.iota(i32, 16)


@functools.partial(
    pl.kernel,
    out_type=jax.ShapeDtypeStruct((NC * 2 * DR + 8, 128), f32),
    mesh=_MESH,
    compiler_params=_SC_PARAMS,
    scratch_types=[
        pltpu.VMEM((2, EW), i32),
        pltpu.VMEM((2 * DR, 128), f32),
        pltpu.VMEM((2, DR), i32),
        pltpu.VMEM((16, 128), f32),
        pltpu.VMEM_SHARED((2 * DR, 128), f32),
    ],
)
def _sc_degrees(e4, deg_out, idx_v, hist, rowidx, zv, shist):
    cid = lax.axis_index("c")
    sid = lax.axis_index("s")
    wid = _wid()
    zero16 = jnp.zeros((16,), f32)
    ones16 = jnp.ones((16,), f32)

    def zbody(i, _):
        for j in range(8):
            hist[i, pl.ds(j * 16, 16)] = zero16
        return 0

    lax.fori_loop(0, 2 * DR, zbody, 0)
    for i in range(16):
        for j in range(8):
            zv[i, pl.ds(j * 16, 16)] = zero16

    for a in range(2):
        for j in range(DR // 16):
            rowidx[a, pl.ds(j * 16, 16)] = _iota16() + (a * DR + j * 16)

    base_r = pl.multiple_of(jnp.minimum(sid * 16, 2 * DR - 16), 16)
    pltpu.sync_copy(zv, shist.at[pl.ds(base_r, 16)])

    pltpu.sync_copy(e4.at[wid], idx_v)

    def hbody(i, _):
        for a in range(2):
            idx = idx_v[a, pl.ds(i * 16, 16)]
            r = (idx >> 7) + (a * DR)
            c = idx & 127
            plsc.addupdate_scatter(hist, [r, c], ones16)
        return 0

    plsc.subcore_barrier()
    lax.fori_loop(0, EW // 16, hbody, 0)

    pltpu.sync_copy(hist.at[pl.ds(0, DR)], shist.at[rowidx.at[0]], add=True)
    pltpu.sync_copy(hist.at[pl.ds(DR, DR)], shist.at[rowidx.at[1]], add=True)
    plsc.subcore_barrier()

    pltpu.sync_copy(
        shist.at[pl.ds(base_r, 16)],
        deg_out.at[pl.ds(cid * (2 * DR) + base_r, 16)],
    )


def _load_deg_windows(deg, base, bufs, which):
    r0 = base >> 7
    r0p = pl.multiple_of(r0 & jnp.int32(-8), 8)
    for (c, a), buf in zip(which, bufs):
        pltpu.sync_copy(deg.at[pl.ds(c * (2 * DR) + a * DR + r0p, 16)], buf)
    return r0p


def _deg16(buf, base, r0p, j):
    n0 = base + j * 16
    row = (n0 >> 7) - r0p
    col = n0 & 127
    return buf[row, pl.ds(col, 16)]


@functools.partial(
    pl.kernel,
    out_type=jax.ShapeDtypeStruct((N, D), f32),
    mesh=_MESH,
    compiler_params=_SC_PARAMS,
    scratch_types=[
        pltpu.VMEM((ROWS_W, D), f32),
        pltpu.VMEM((16, 128), f32),
        pltpu.VMEM((16, 128), f32),
        pltpu.VMEM((ROWS_W,), f32),
    ],
)
def _sc_scale_src(x, deg, xs_out, xbuf, w0, w1, nrm):
    wid = _wid()
    base = pl.multiple_of(jnp.minimum(wid * ROWS_W, N - ROWS_W), 16)
    pltpu.sync_copy(x.at[pl.ds(base, ROWS_W)], xbuf)
    r0p = _load_deg_windows(deg, base, [w0, w1], [(0, 0), (1, 0)])

    def nbody(j, _):
        d0 = _deg16(w0, base, r0p, j)
        d1 = _deg16(w1, base, r0p, j)
        nrm[pl.ds(j * 16, 16)] = _norm16(d0, d1)
        return 0

    lax.fori_loop(0, ROWS_W // 16, nbody, 0)

    def rbody(g, _):
        nv = nrm[pl.ds(g * 16, 16)]
        for k in range(16):
            r = g * 16 + k
            sv = jnp.full((16,), nv[k], dtype=f32)
            for j in range(D // 16):
                sl = pl.ds(j * 16, 16)
                xbuf[r, sl] = xbuf[r, sl] * sv
        return 0

    lax.fori_loop(0, ROWS_W // 16, rbody, 0)
    pltpu.sync_copy(xbuf, xs_out.at[pl.ds(base, ROWS_W)])


def _mm_body(a_ref, w_ref, o_ref):
    o_ref[...] = jnp.dot(a_ref[...], w_ref[...], preferred_element_type=f32)


def _tc_matmul(a, w):
    return pl.pallas_call(
        _mm_body,
        grid=(N // 400,),
        in_specs=[
            pl.BlockSpec((400, D), lambda i: (i, 0)),
            pl.BlockSpec((D, D), lambda i: (0, 0)),
        ],
        out_specs=pl.BlockSpec((400, D), lambda i: (i, 0)),
        out_shape=jax.ShapeDtypeStruct((N, D), f32),
    )(a, w)


@functools.partial(
    pl.kernel,
    out_type=jax.ShapeDtypeStruct((NC * N, D), f32),
    mesh=_MESH,
    compiler_params=_SC_PARAMS,
    scratch_types=[
        pltpu.VMEM((NCH, CH), i32),
        pltpu.VMEM((NCH, CH), i32),
        pltpu.VMEM((CH, D), f32),
        pltpu.VMEM_SHARED((N, D), f32),
        pltpu.SemaphoreType.DMA,
    ],
)
def _sc_agg(z, ecs, zrows, part_out, srcv, dstv, rows, acc, sem):
    cid = lax.axis_index("c")
    sid = lax.axis_index("s")
    wid = _wid()
    base_t = pl.multiple_of(jnp.minimum(sid * ROWS_T, N - ROWS_T), 16)
    pltpu.sync_copy(zrows, acc.at[pl.ds(base_t, ROWS_T)])
    pltpu.sync_copy(ecs.at[pl.ds(wid * 2 * NCH, NCH)], srcv)
    pltpu.sync_copy(ecs.at[pl.ds(wid * 2 * NCH + NCH, NCH)], dstv)
    plsc.subcore_barrier()

    def body(c, _):
        pltpu.async_copy(z.at[srcv.at[c]], rows, sem).wait()
        pltpu.sync_copy(rows, acc.at[dstv.at[c]], add=True)
        return 0

    lax.fori_loop(0, NCH, body, 0)
    plsc.subcore_barrier()
    pltpu.sync_copy(
        acc.at[pl.ds(base_t, ROWS_T)],
        part_out.at[pl.ds(cid * N + base_t, ROWS_T)],
    )


def _fuse_body(p, deg, bias, out, pbuf0, pbuf1, w0, w1, ns, nd, bvec, *, mid):
    wid = _wid()
    base = pl.multiple_of(jnp.minimum(wid * ROWS_W, N - ROWS_W), 16)
    pltpu.sync_copy(p.at[pl.ds(base, ROWS_W)], pbuf0)
    pltpu.sync_copy(p.at[pl.ds(N + base, ROWS_W)], pbuf1)
    pltpu.sync_copy(bias, bvec)
    r0p = _load_deg_windows(deg, base, [w0, w1], [(0, 1), (1, 1)])

    def ndbody(j, _):
        d0 = _deg16(w0, base, r0p, j)
        d1 = _deg16(w1, base, r0p, j)
        nd[pl.ds(j * 16, 16)] = _norm16(d0, d1)
        return 0

    lax.fori_loop(0, ROWS_W // 16, ndbody, 0)
    if mid:
        _load_deg_windows(deg, base, [w0, w1], [(0, 0), (1, 0)])

        def nsbody(j, _):
            d0 = _deg16(w0, base, r0p, j)
            d1 = _deg16(w1, base, r0p, j)
            ns[pl.ds(j * 16, 16)] = _norm16(d0, d1)
            return 0

        lax.fori_loop(0, ROWS_W // 16, nsbody, 0)

    def rbody(g, _):
        ndg = nd[pl.ds(g * 16, 16)]
        if mid:
            nsg = ns[pl.ds(g * 16, 16)]
        for k in range(16):
            r = g * 16 + k
            ndv = jnp.full((16,), ndg[k], dtype=f32)
            if mid:
                nsv = jnp.full((16,), nsg[k], dtype=f32)
            for j in range(D // 16):
                sl = pl.ds(j * 16, 16)
                v = (pbuf0[r, sl] + pbuf1[r, sl]) * ndv + bvec[sl]
                if mid:
                    v = jnp.maximum(v, 0.0) * nsv
                pbuf0[r, sl] = v
        return 0

    lax.fori_loop(0, ROWS_W // 16, rbody, 0)
    pltpu.sync_copy(pbuf0, out.at[pl.ds(base, ROWS_W)])


def _fuse_scratch():
    return [
        pltpu.VMEM((ROWS_W, D), f32),
        pltpu.VMEM((ROWS_W, D), f32),
        pltpu.VMEM((16, 128), f32),
        pltpu.VMEM((16, 128), f32),
        pltpu.VMEM((ROWS_W,), f32),
        pltpu.VMEM((ROWS_W,), f32),
        pltpu.VMEM((D,), f32),
    ]


_sc_fuse_mid = functools.partial(
    pl.kernel,
    out_type=jax.ShapeDtypeStruct((N, D), f32),
    mesh=_MESH,
    compiler_params=_SC_PARAMS,
    scratch_types=_fuse_scratch(),
)(functools.partial(_fuse_body, mid=True))

_sc_fuse_out = functools.partial(
    pl.kernel,
    out_type=jax.ShapeDtypeStruct((N, D), f32),
    mesh=_MESH,
    compiler_params=_SC_PARAMS,
    scratch_types=_fuse_scratch(),
)(functools.partial(_fuse_body, mid=False))


def kernel(x, edge_index, W1, b1, W2, b2):
    e4 = edge_index.reshape(2, NW, EW).transpose(1, 0, 2)
    ecs = (
        edge_index.reshape(2, NW, NCH, CH)
        .transpose(1, 0, 2, 3)
        .reshape(NW * 2 * NCH, CH)
    )
    zrows = jnp.zeros((ROWS_T, D), dtype=f32)

    deg = _sc_degrees(e4)
    xs = _sc_scale_src(x, deg)
    z1 = _tc_matmul(xs, W1)
    p1 = _sc_agg(z1, ecs, zrows)
    s2 = _sc_fuse_mid(p1, deg, b1)
    z2 = _tc_matmul(s2, W2)
    p2 = _sc_agg(z2, ecs, zrows)
    return _sc_fuse_out(p2, deg, b2)


# speedup vs baseline: 10.9424x; 1.3594x over previous
import functools

import jax
import jax.numpy as jnp
from jax import lax
from jax.experimental import pallas as pl
from jax.experimental.pallas import tpu as pltpu
from jax.experimental.pallas import tpu_sc as plsc

N = 10000
E = 320000
D = 128
NC = 2
NS = 16
NW = NC * NS
EW = E // NW
CH = 125
NCH = EW // CH
ROWS_W = 320
DR = 80
ROWS_T = 640

_MESH = plsc.VectorSubcoreMesh(core_axis_name="c", subcore_axis_name="s")
_SC_PARAMS = pltpu.CompilerParams(needs_layout_passes=False)

f32 = jnp.float32
i32 = jnp.int32


def _wid():
    return lax.axis_index("s") * NC + lax.axis_index("c")


def _rsqrt16(x):
    i = plsc.bitcast(x, i32)
    i = jnp.int32(0x5F3759DF) - (i >> 1)
    y = plsc.bitcast(i, f32)
    for _ in range(3):
        y = y * (1.5 - 0.5 * x * y * y)
    return y


def _norm16(d0, d1):
    return _rsqrt16(jnp.maximum(d0 + d1, 1.0))


def _iota16():
    return lax.iota(i32, 16)


@functools.partial(
    pl.kernel,
    out_type=jax.ShapeDtypeStruct((NC * 2 * DR + 8, 128), f32),
    mesh=_MESH,
    compiler_params=_SC_PARAMS,
    scratch_types=[
        pltpu.VMEM((2, EW), i32),
        pltpu.VMEM((2 * DR, 128), f32),
        pltpu.VMEM((2, DR), i32),
        pltpu.VMEM((16, 128), f32),
        pltpu.VMEM_SHARED((2 * DR, 128), f32),
    ],
)
def _sc_degrees(e4, deg_out, idx_v, hist, rowidx, zv, shist):
    cid = lax.axis_index("c")
    sid = lax.axis_index("s")
    wid = _wid()
    zero16 = jnp.zeros((16,), f32)
    ones16 = jnp.ones((16,), f32)

    def zbody(i, _):
        for j in range(8):
            hist[i, pl.ds(j * 16, 16)] = zero16
        return 0

    lax.fori_loop(0, 2 * DR, zbody, 0)
    for i in range(16):
        for j in range(8):
            zv[i, pl.ds(j * 16, 16)] = zero16

    for a in range(2):
        for j in range(DR // 16):
            rowidx[a, pl.ds(j * 16, 16)] = _iota16() + (a * DR + j * 16)

    base_r = pl.multiple_of(jnp.minimum(sid * 16, 2 * DR - 16), 16)
    pltpu.sync_copy(zv, shist.at[pl.ds(base_r, 16)])

    pltpu.sync_copy(e4.at[wid], idx_v)

    def hbody(i, _):
        for a in range(2):
            idx = idx_v[a, pl.ds(i * 16, 16)]
            r = (idx >> 7) + (a * DR)
            c = idx & 127
            plsc.addupdate_scatter(hist, [r, c], ones16)
        return 0

    plsc.subcore_barrier()
    lax.fori_loop(0, EW // 16, hbody, 0)

    pltpu.sync_copy(hist.at[pl.ds(0, DR)], shist.at[rowidx.at[0]], add=True)
    pltpu.sync_copy(hist.at[pl.ds(DR, DR)], shist.at[rowidx.at[1]], add=True)
    plsc.subcore_barrier()

    pltpu.sync_copy(
        shist.at[pl.ds(base_r, 16)],
        deg_out.at[pl.ds(cid * (2 * DR) + base_r, 16)],
    )


def _load_deg_windows(deg, base, bufs, which):
    r0 = base >> 7
    r0p = pl.multiple_of(r0 & jnp.int32(-8), 8)
    for (c, a), buf in zip(which, bufs):
        pltpu.sync_copy(deg.at[pl.ds(c * (2 * DR) + a * DR + r0p, 16)], buf)
    return r0p


def _deg16(buf, base, r0p, j):
    n0 = base + j * 16
    row = (n0 >> 7) - r0p
    col = n0 & 127
    return buf[row, pl.ds(col, 16)]


@functools.partial(
    pl.kernel,
    out_type=jax.ShapeDtypeStruct((N, D), f32),
    mesh=_MESH,
    compiler_params=_SC_PARAMS,
    scratch_types=[
        pltpu.VMEM((ROWS_W, D), f32),
        pltpu.VMEM((16, 128), f32),
        pltpu.VMEM((16, 128), f32),
        pltpu.VMEM((ROWS_W,), f32),
    ],
)
def _sc_scale_src(x, deg, xs_out, xbuf, w0, w1, nrm):
    wid = _wid()
    base = pl.multiple_of(jnp.minimum(wid * ROWS_W, N - ROWS_W), 16)
    pltpu.sync_copy(x.at[pl.ds(base, ROWS_W)], xbuf)
    r0p = _load_deg_windows(deg, base, [w0, w1], [(0, 0), (1, 0)])

    def nbody(j, _):
        d0 = _deg16(w0, base, r0p, j)
        d1 = _deg16(w1, base, r0p, j)
        nrm[pl.ds(j * 16, 16)] = _norm16(d0, d1)
        return 0

    lax.fori_loop(0, ROWS_W // 16, nbody, 0)

    def rbody(g, _):
        nv = nrm[pl.ds(g * 16, 16)]
        for k in range(16):
            r = g * 16 + k
            sv = jnp.full((16,), nv[k], dtype=f32)
            for j in range(D // 16):
                sl = pl.ds(j * 16, 16)
                xbuf[r, sl] = xbuf[r, sl] * sv
        return 0

    lax.fori_loop(0, ROWS_W // 16, rbody, 0)
    pltpu.sync_copy(xbuf, xs_out.at[pl.ds(base, ROWS_W)])


def _mm_body(a_ref, w_ref, o_ref):
    o_ref[...] = jnp.dot(a_ref[...], w_ref[...], preferred_element_type=f32)


def _tc_matmul(a, w):
    return pl.pallas_call(
        _mm_body,
        grid=(N // 400,),
        in_specs=[
            pl.BlockSpec((400, D), lambda i: (i, 0)),
            pl.BlockSpec((D, D), lambda i: (0, 0)),
        ],
        out_specs=pl.BlockSpec((400, D), lambda i: (i, 0)),
        out_shape=jax.ShapeDtypeStruct((N, D), f32),
    )(a, w)


@functools.partial(
    pl.kernel,
    out_type=jax.ShapeDtypeStruct((NC * N, D), f32),
    mesh=_MESH,
    compiler_params=_SC_PARAMS,
    scratch_types=[
        pltpu.VMEM((NCH // 2, CH), i32),
        pltpu.VMEM((NCH // 2, CH), i32),
        pltpu.VMEM((CH, D), f32),
        pltpu.VMEM((CH, D), f32),
        pltpu.VMEM_SHARED((N, D), f32),
        pltpu.SemaphoreType.DMA,
        pltpu.SemaphoreType.DMA,
    ],
)
def _sc_agg(z, ecs, zrows, part_out, srcv, dstv, rows0, rows1, acc, sem0, sem1):
    cid = lax.axis_index("c")
    sid = lax.axis_index("s")
    wid = _wid()
    hch = NCH // 2
    base_t = pl.multiple_of(jnp.minimum(sid * ROWS_T, N - ROWS_T), 16)
    pltpu.sync_copy(zrows, acc.at[pl.ds(base_t, ROWS_T)])
    plsc.subcore_barrier()

    for h in range(2):
        pltpu.sync_copy(ecs.at[pl.ds(wid * 2 * NCH + h * hch, hch)], srcv)
        pltpu.sync_copy(ecs.at[pl.ds(wid * 2 * NCH + NCH + h * hch, hch)], dstv)
        pltpu.async_copy(z.at[srcv.at[0]], rows0, sem0)

        def body(i, _):
            c0 = i * 2
            c1 = i * 2 + 1
            pltpu.async_copy(z.at[srcv.at[c1]], rows1, sem1)
            pltpu.make_async_copy(z.at[srcv.at[c0]], rows0, sem0).wait()
            pltpu.sync_copy(rows0, acc.at[dstv.at[c0]], add=True)
            cn = jnp.minimum(c0 + 2, hch - 1)
            pltpu.async_copy(z.at[srcv.at[cn]], rows0, sem0)
            pltpu.make_async_copy(z.at[srcv.at[c1]], rows1, sem1).wait()
            pltpu.sync_copy(rows1, acc.at[dstv.at[c1]], add=True)
            return 0

        lax.fori_loop(0, hch // 2, body, 0)
        pltpu.make_async_copy(z.at[srcv.at[0]], rows0, sem0).wait()
    plsc.subcore_barrier()
    pltpu.sync_copy(
        acc.at[pl.ds(base_t, ROWS_T)],
        part_out.at[pl.ds(cid * N + base_t, ROWS_T)],
    )


def _fuse_body(p, deg, bias, out, pbuf0, pbuf1, w0, w1, ns, nd, bvec, *, mid):
    wid = _wid()
    base = pl.multiple_of(jnp.minimum(wid * ROWS_W, N - ROWS_W), 16)
    pltpu.sync_copy(p.at[pl.ds(base, ROWS_W)], pbuf0)
    pltpu.sync_copy(p.at[pl.ds(N + base, ROWS_W)], pbuf1)
    pltpu.sync_copy(bias, bvec)
    r0p = _load_deg_windows(deg, base, [w0, w1], [(0, 1), (1, 1)])

    def ndbody(j, _):
        d0 = _deg16(w0, base, r0p, j)
        d1 = _deg16(w1, base, r0p, j)
        nd[pl.ds(j * 16, 16)] = _norm16(d0, d1)
        return 0

    lax.fori_loop(0, ROWS_W // 16, ndbody, 0)
    if mid:
        _load_deg_windows(deg, base, [w0, w1], [(0, 0), (1, 0)])

        def nsbody(j, _):
            d0 = _deg16(w0, base, r0p, j)
            d1 = _deg16(w1, base, r0p, j)
            ns[pl.ds(j * 16, 16)] = _norm16(d0, d1)
            return 0

        lax.fori_loop(0, ROWS_W // 16, nsbody, 0)

    def rbody(g, _):
        ndg = nd[pl.ds(g * 16, 16)]
        if mid:
            nsg = ns[pl.ds(g * 16, 16)]
        for k in range(16):
            r = g * 16 + k
            ndv = jnp.full((16,), ndg[k], dtype=f32)
            if mid:
                nsv = jnp.full((16,), nsg[k], dtype=f32)
            for j in range(D // 16):
                sl = pl.ds(j * 16, 16)
                v = (pbuf0[r, sl] + pbuf1[r, sl]) * ndv + bvec[sl]
                if mid:
                    v = jnp.maximum(v, 0.0) * nsv
                pbuf0[r, sl] = v
        return 0

    lax.fori_loop(0, ROWS_W // 16, rbody, 0)
    pltpu.sync_copy(pbuf0, out.at[pl.ds(base, ROWS_W)])


def _fuse_scratch():
    return [
        pltpu.VMEM((ROWS_W, D), f32),
        pltpu.VMEM((ROWS_W, D), f32),
        pltpu.VMEM((16, 128), f32),
        pltpu.VMEM((16, 128), f32),
        pltpu.VMEM((ROWS_W,), f32),
        pltpu.VMEM((ROWS_W,), f32),
        pltpu.VMEM((D,), f32),
    ]


_sc_fuse_mid = functools.partial(
    pl.kernel,
    out_type=jax.ShapeDtypeStruct((N, D), f32),
    mesh=_MESH,
    compiler_params=_SC_PARAMS,
    scratch_types=_fuse_scratch(),
)(functools.partial(_fuse_body, mid=True))

_sc_fuse_out = functools.partial(
    pl.kernel,
    out_type=jax.ShapeDtypeStruct((N, D), f32),
    mesh=_MESH,
    compiler_params=_SC_PARAMS,
    scratch_types=_fuse_scratch(),
)(functools.partial(_fuse_body, mid=False))


def kernel(x, edge_index, W1, b1, W2, b2):
    e4 = edge_index.reshape(2, NW, EW).transpose(1, 0, 2)
    ecs = (
        edge_index.reshape(2, NW, NCH, CH)
        .transpose(1, 0, 2, 3)
        .reshape(NW * 2 * NCH, CH)
    )
    zrows = jnp.zeros((ROWS_T, D), dtype=f32)

    deg = _sc_degrees(e4)
    xs = _sc_scale_src(x, deg)
    z1 = _tc_matmul(xs, W1)
    p1 = _sc_agg(z1, ecs, zrows)
    s2 = _sc_fuse_mid(p1, deg, b1)
    z2 = _tc_matmul(s2, W2)
    p2 = _sc_agg(z2, ecs, zrows)
    return _sc_fuse_out(p2, deg, b2)
